# baseline (speedup 1.0000x reference)
import jax
import jax.numpy as jnp
from jax import lax
from jax.experimental import pallas as pl
from jax.experimental.pallas import tpu as pltpu
from jax.experimental.pallas import tpu_sc as plsc

ENT_SIZE = 64
REL_SIZE = 64
MARGIN = 1.0
BATCH = 4096
NC, NS = 2, 16
NW = NC * NS
BPW = BATCH // NW
CHUNK = 8
NCHUNK = BPW // CHUNK
L = 16
EV = ENT_SIZE // L


def _rsqrt_nr(x):
    xh = x * 0.5
    i = lax.bitcast_convert_type(x, jnp.int32)
    i = jnp.int32(0x5F3759DF) - lax.shift_right_logical(i, 1)
    y = lax.bitcast_convert_type(i, jnp.float32)
    for _ in range(3):
        y = y * (1.5 - xh * y * y)
    return y


def _lanesum(x, ii):
    for sh in (8, 4, 2, 1):
        x = x + jnp.take(x, ii ^ sh)
    return x


def _l2n(vs, ii):
    ss = vs[0] * vs[0]
    for j in range(1, EV):
        ss = ss + vs[j] * vs[j]
    total = _lanesum(ss, ii)
    inv = _rsqrt_nr(jnp.maximum(total, 1e-12))
    return [v * inv for v in vs]


def _sc_body(ent_hbm, rel_hbm, tm_hbm,
             ph_hbm, pt_hbm, pr_hbm, nh_hbm, nt_hbm, nr_hbm,
             out_hbm,
             iv_ph, iv_pt, iv_pr, iv_nh, iv_nt, iv_nr,
             e_ph, e_pt, e_nh, e_nt, e_pr, e_nr,
             mbuf, outv, sem):
    wid = lax.axis_index("s") * NC + lax.axis_index("c")
    base = wid * BPW
    ii = lax.iota(jnp.int32, L)

    pltpu.sync_copy(ph_hbm.at[pl.ds(base, BPW)], iv_ph)
    pltpu.sync_copy(pt_hbm.at[pl.ds(base, BPW)], iv_pt)
    pltpu.sync_copy(pr_hbm.at[pl.ds(base, BPW)], iv_pr)
    pltpu.sync_copy(nh_hbm.at[pl.ds(base, BPW)], iv_nh)
    pltpu.sync_copy(nt_hbm.at[pl.ds(base, BPW)], iv_nt)
    pltpu.sync_copy(nr_hbm.at[pl.ds(base, BPW)], iv_nr)

    c1 = pltpu.async_copy(ent_hbm.at[iv_ph], e_ph, sem)
    c2 = pltpu.async_copy(ent_hbm.at[iv_pt], e_pt, sem)
    c3 = pltpu.async_copy(ent_hbm.at[iv_nh], e_nh, sem)
    c4 = pltpu.async_copy(ent_hbm.at[iv_nt], e_nt, sem)
    c5 = pltpu.async_copy(rel_hbm.at[iv_pr], e_pr, sem)
    c6 = pltpu.async_copy(rel_hbm.at[iv_nr], e_nr, sem)
    c1.wait(); c2.wait(); c3.wait(); c4.wait(); c5.wait(); c6.wait()

    def sample(b, s, acc):
        zeros = [jnp.zeros((L,), jnp.float32) for _ in range(EV)]

        def ebody(eo, carry):
            ph, pt, nh, nt = carry
            hv = e_ph[b, pl.ds(eo * L, L)]
            tv = e_pt[b, pl.ds(eo * L, L)]
            nhv = e_nh[b, pl.ds(eo * L, L)]
            ntv = e_nt[b, pl.ds(eo * L, L)]
            off0 = eo * (L * REL_SIZE)
            for el in range(L):
                m = [mbuf[s, pl.ds(off0 + el * REL_SIZE + j * L, L)]
                     for j in range(EV)]
                hb = jnp.full((L,), hv[el])
                tb = jnp.full((L,), tv[el])
                nhb = jnp.full((L,), nhv[el])
                ntb = jnp.full((L,), ntv[el])
                ph = tuple(ph[j] + hb * m[j] for j in range(EV))
                pt = tuple(pt[j] + tb * m[j] for j in range(EV))
                nh = tuple(nh[j] + nhb * m[j] for j in range(EV))
                nt = tuple(nt[j] + ntb * m[j] for j in range(EV))
            return ph, pt, nh, nt

        init = (tuple(zeros), tuple(zeros), tuple(zeros), tuple(zeros))
        ph, pt, nh, nt = lax.fori_loop(0, EV, ebody, init)

        phn = _l2n(list(ph), ii)
        ptn = _l2n(list(pt), ii)
        nhn = _l2n(list(nh), ii)
        ntn = _l2n(list(nt), ii)
        prn = _l2n([e_pr[b, pl.ds(j * L, L)] for j in range(EV)], ii)
        nrn = _l2n([e_nr[b, pl.ds(j * L, L)] for j in range(EV)], ii)

        pd = jnp.abs(phn[0] + prn[0] - ptn[0])
        nd = jnp.abs(nhn[0] + nrn[0] - ntn[0])
        for j in range(1, EV):
            pd = pd + jnp.abs(phn[j] + prn[j] - ptn[j])
            nd = nd + jnp.abs(nhn[j] + nrn[j] - ntn[j])
        p_sc = _lanesum(pd, ii)
        n_sc = _lanesum(nd, ii)
        return acc + jnp.maximum(p_sc - n_sc + MARGIN, 0.0)

    def chunk_body(c, acc):
        idx8 = iv_pr.at[pl.ds(c * CHUNK, CHUNK)]
        pltpu.async_copy(tm_hbm.at[idx8], mbuf, sem).wait()
        return lax.fori_loop(
            0, CHUNK, lambda s, a: sample(c * CHUNK + s, s, a), acc)

    acc = lax.fori_loop(0, NCHUNK, chunk_body, jnp.zeros((L,), jnp.float32))
    outv[...] = acc
    pltpu.sync_copy(outv, out_hbm.at[wid])


def kernel(ent_embeddings, rel_embeddings, transfer_matrix,
           pos_h, pos_t, pos_r, neg_h, neg_t, neg_r):
    idx = [x.astype(jnp.int32) for x in (pos_h, pos_t, pos_r,
                                         neg_h, neg_t, neg_r)]
    mesh = plsc.VectorSubcoreMesh(core_axis_name="c", subcore_axis_name="s")
    run = pl.kernel(
        _sc_body,
        out_type=jax.ShapeDtypeStruct((NW, L), jnp.float32),
        mesh=mesh,
        compiler_params=pltpu.CompilerParams(use_tc_tiling_on_sc=False),
        scratch_types=[
            pltpu.VMEM((BPW,), jnp.int32),
            pltpu.VMEM((BPW,), jnp.int32),
            pltpu.VMEM((BPW,), jnp.int32),
            pltpu.VMEM((BPW,), jnp.int32),
            pltpu.VMEM((BPW,), jnp.int32),
            pltpu.VMEM((BPW,), jnp.int32),
            pltpu.VMEM((BPW, ENT_SIZE), jnp.float32),
            pltpu.VMEM((BPW, ENT_SIZE), jnp.float32),
            pltpu.VMEM((BPW, ENT_SIZE), jnp.float32),
            pltpu.VMEM((BPW, ENT_SIZE), jnp.float32),
            pltpu.VMEM((BPW, REL_SIZE), jnp.float32),
            pltpu.VMEM((BPW, REL_SIZE), jnp.float32),
            pltpu.VMEM((CHUNK, ENT_SIZE * REL_SIZE), jnp.float32),
            pltpu.VMEM((L,), jnp.float32),
            pltpu.SemaphoreType.DMA,
        ],
    )
    partials = run(ent_embeddings, rel_embeddings, transfer_matrix, *idx)
    return jnp.sum(partials[:, 0]) / BATCH


# speedup vs baseline: 1.5292x; 1.5292x over previous
import jax
import jax.numpy as jnp
from jax import lax
from jax.experimental import pallas as pl
from jax.experimental.pallas import tpu as pltpu
from jax.experimental.pallas import tpu_sc as plsc

ENT_SIZE = 64
REL_SIZE = 64
MARGIN = 1.0
BATCH = 4096
NC, NS = 2, 16
NW = NC * NS
BPW = BATCH // NW
CHUNK = 8
NCHUNK = BPW // CHUNK
L = 16
EV = ENT_SIZE // L
OUTW = 128


def _rsqrt_nr(x):
    xh = x * 0.5
    i = lax.bitcast_convert_type(x, jnp.int32)
    i = jnp.int32(0x5F3759DF) - lax.shift_right_logical(i, 1)
    y = lax.bitcast_convert_type(i, jnp.float32)
    for _ in range(3):
        y = y * (1.5 - xh * y * y)
    return y


def _lanesum(x, ii):
    for sh in (8, 4, 2, 1):
        x = x + jnp.take(x, ii ^ sh)
    return x


def _l2n(vs, ii):
    ss = vs[0] * vs[0]
    for j in range(1, EV):
        ss = ss + vs[j] * vs[j]
    total = _lanesum(ss, ii)
    inv = _rsqrt_nr(jnp.maximum(total, 1e-12))
    return [v * inv for v in vs]


def _sc_body(ent_hbm, rel_hbm, tm_hbm,
             ph_hbm, pt_hbm, pr_hbm, nh_hbm, nt_hbm, nr_hbm,
             out_hbm,
             iv_ph, iv_pt, iv_pr, iv_nh, iv_nt, iv_nr,
             eb_ph0, eb_pt0, eb_nh0, eb_nt0, eb_pr0, eb_nr0,
             eb_ph1, eb_pt1, eb_nh1, eb_nt1, eb_pr1, eb_nr1,
             mbuf0, mbuf1, outv,
             msem0, msem1, esem0, esem1):
    wid = lax.axis_index("s") * NC + lax.axis_index("c")
    base = wid * BPW
    ii = lax.iota(jnp.int32, L)

    ebufs0 = (eb_ph0, eb_pt0, eb_nh0, eb_nt0, eb_pr0, eb_nr0)
    ebufs1 = (eb_ph1, eb_pt1, eb_nh1, eb_nt1, eb_pr1, eb_nr1)
    ivlist = (iv_ph, iv_pt, iv_nh, iv_nt, iv_pr, iv_nr)
    srcs = (ent_hbm, ent_hbm, ent_hbm, ent_hbm, rel_hbm, rel_hbm)

    pltpu.sync_copy(ph_hbm.at[pl.ds(base, BPW)], iv_ph)
    pltpu.sync_copy(pt_hbm.at[pl.ds(base, BPW)], iv_pt)
    pltpu.sync_copy(pr_hbm.at[pl.ds(base, BPW)], iv_pr)
    pltpu.sync_copy(nh_hbm.at[pl.ds(base, BPW)], iv_nh)
    pltpu.sync_copy(nt_hbm.at[pl.ds(base, BPW)], iv_nt)
    pltpu.sync_copy(nr_hbm.at[pl.ds(base, BPW)], iv_nr)

    def issue(c, half, ebufs, mbuf, msem, esem):
        off = (c // 2) * L
        pltpu.async_copy(tm_hbm.at[iv_pr.at[pl.ds(c * CHUNK, CHUNK)]],
                         mbuf, msem)
        for src, ebuf, ivref in zip(srcs, ebufs, ivlist):
            vec = ivref[pl.ds(off, L)]
            for s in range(CHUNK):
                idx = vec[half * CHUNK + s]
                pltpu.async_copy(src.at[idx], ebuf.at[s], esem)

    def drain(ebufs, mbuf, msem, esem):
        pltpu.make_async_copy(tm_hbm.at[pl.ds(0, CHUNK)], mbuf, msem).wait()
        for src, ebuf in zip(srcs, ebufs):
            pltpu.make_async_copy(src.at[pl.ds(0, CHUNK)], ebuf, esem).wait()

    def compute(ebufs, mbuf, acc):
        eb_h, eb_t, eb_nh, eb_nt, eb_r, eb_nr = ebufs

        def sample(s, acc):
            zeros = [jnp.zeros((L,), jnp.float32) for _ in range(EV)]

            def ebody(eo, carry):
                ph, pt, nh, nt = carry
                hv = eb_h[s, pl.ds(eo * L, L)]
                tv = eb_t[s, pl.ds(eo * L, L)]
                nhv = eb_nh[s, pl.ds(eo * L, L)]
                ntv = eb_nt[s, pl.ds(eo * L, L)]
                off0 = eo * (L * REL_SIZE)
                for el in range(L):
                    m = [mbuf[s, pl.ds(off0 + el * REL_SIZE + j * L, L)]
                         for j in range(EV)]
                    hb = jnp.full((L,), hv[el])
                    tb = jnp.full((L,), tv[el])
                    nhb = jnp.full((L,), nhv[el])
                    ntb = jnp.full((L,), ntv[el])
                    ph = tuple(ph[j] + hb * m[j] for j in range(EV))
                    pt = tuple(pt[j] + tb * m[j] for j in range(EV))
                    nh = tuple(nh[j] + nhb * m[j] for j in range(EV))
                    nt = tuple(nt[j] + ntb * m[j] for j in range(EV))
                return ph, pt, nh, nt

            init = (tuple(zeros), tuple(zeros), tuple(zeros), tuple(zeros))
            ph, pt, nh, nt = lax.fori_loop(0, EV, ebody, init)

            phn = _l2n(list(ph), ii)
            ptn = _l2n(list(pt), ii)
            nhn = _l2n(list(nh), ii)
            ntn = _l2n(list(nt), ii)
            prn = _l2n([eb_r[s, pl.ds(j * L, L)] for j in range(EV)], ii)
            nrn = _l2n([eb_nr[s, pl.ds(j * L, L)] for j in range(EV)], ii)

            pd = jnp.abs(phn[0] + prn[0] - ptn[0])
            nd = jnp.abs(nhn[0] + nrn[0] - ntn[0])
            for j in range(1, EV):
                pd = pd + jnp.abs(phn[j] + prn[j] - ptn[j])
                nd = nd + jnp.abs(nhn[j] + nrn[j] - ntn[j])
            p_sc = _lanesum(pd, ii)
            n_sc = _lanesum(nd, ii)
            return acc + jnp.maximum(p_sc - n_sc + MARGIN, 0.0)

        return lax.fori_loop(0, CHUNK, sample, acc)

    issue(0, 0, ebufs0, mbuf0, msem0, esem0)

    def body(i, acc):
        c0 = 2 * i
        issue(c0 + 1, 1, ebufs1, mbuf1, msem1, esem1)
        drain(ebufs0, mbuf0, msem0, esem0)
        acc = compute(ebufs0, mbuf0, acc)

        @pl.when(i < NCHUNK // 2 - 1)
        def _():
            issue(c0 + 2, 0, ebufs0, mbuf0, msem0, esem0)

        drain(ebufs1, mbuf1, msem1, esem1)
        return compute(ebufs1, mbuf1, acc)

    acc = lax.fori_loop(0, NCHUNK // 2, body, jnp.zeros((L,), jnp.float32))
    outv[pl.ds(0, L)] = acc
    pltpu.sync_copy(outv, out_hbm.at[wid])


def kernel(ent_embeddings, rel_embeddings, transfer_matrix,
           pos_h, pos_t, pos_r, neg_h, neg_t, neg_r):
    idx = [x.astype(jnp.int32) for x in (pos_h, pos_t, pos_r,
                                         neg_h, neg_t, neg_r)]
    mesh = plsc.VectorSubcoreMesh(core_axis_name="c", subcore_axis_name="s")
    ebuf = pltpu.VMEM((CHUNK, ENT_SIZE), jnp.float32)
    run = pl.kernel(
        _sc_body,
        out_type=jax.ShapeDtypeStruct((NW, OUTW), jnp.float32),
        mesh=mesh,
        scratch_types=(
            [pltpu.VMEM((BPW,), jnp.int32)] * 6
            + [ebuf] * 12
            + [pltpu.VMEM((CHUNK, ENT_SIZE * REL_SIZE), jnp.float32)] * 2
            + [pltpu.VMEM((OUTW,), jnp.float32)]
            + [pltpu.SemaphoreType.DMA] * 4
        ),
    )
    partials = run(ent_embeddings, rel_embeddings, transfer_matrix, *idx)
    return jnp.sum(partials[:, 0]) / BATCH


# speedup vs baseline: 1.8581x; 1.2151x over previous
import jax
import jax.numpy as jnp
from jax import lax
from jax.experimental import pallas as pl
from jax.experimental.pallas import tpu as pltpu
from jax.experimental.pallas import tpu_sc as plsc

ENT_SIZE = 64
REL_SIZE = 64
MARGIN = 1.0
BATCH = 4096
NC, NS = 2, 16
NW = NC * NS
BPW = BATCH // NW
CHUNK = 8
NCHUNK = BPW // CHUNK
L = 16
EV = ENT_SIZE // L
OUTW = 128


def _rsqrt_nr(x):
    xh = x * 0.5
    i = lax.bitcast_convert_type(x, jnp.int32)
    i = jnp.int32(0x5F3759DF) - lax.shift_right_logical(i, 1)
    y = lax.bitcast_convert_type(i, jnp.float32)
    for _ in range(3):
        y = y * (1.5 - xh * y * y)
    return y


def _lanesum(x, ii):
    for sh in (8, 4, 2, 1):
        x = x + jnp.take(x, ii ^ sh)
    return x


def _l2n(vs, ii):
    ss = vs[0] * vs[0]
    for j in range(1, EV):
        ss = ss + vs[j] * vs[j]
    total = _lanesum(ss, ii)
    inv = _rsqrt_nr(jnp.maximum(total, 1e-12))
    return [v * inv for v in vs]


def _sc_body(tm_hbm, he_hbm, te_hbm, nhe_hbm, nte_hbm, re_hbm, nre_hbm,
             pr_hbm,
             out_hbm,
             iv_pr,
             eb_h0, eb_t0, eb_nh0, eb_nt0, eb_r0, eb_nr0,
             eb_h1, eb_t1, eb_nh1, eb_nt1, eb_r1, eb_nr1,
             mbuf0, mbuf1, outv,
             msem0, msem1, esem0, esem1):
    wid = lax.axis_index("s") * NC + lax.axis_index("c")
    base = wid * BPW
    ii = lax.iota(jnp.int32, L)

    ebufs0 = (eb_h0, eb_t0, eb_nh0, eb_nt0, eb_r0, eb_nr0)
    ebufs1 = (eb_h1, eb_t1, eb_nh1, eb_nt1, eb_r1, eb_nr1)
    rows_hbm = (he_hbm, te_hbm, nhe_hbm, nte_hbm, re_hbm, nre_hbm)

    pltpu.sync_copy(pr_hbm.at[pl.ds(base, BPW)], iv_pr)

    def issue(c, ebufs, mbuf, msem, esem):
        pltpu.async_copy(tm_hbm.at[iv_pr.at[pl.ds(c * CHUNK, CHUNK)]],
                         mbuf, msem)
        for src, ebuf in zip(rows_hbm, ebufs):
            pltpu.async_copy(src.at[pl.ds(base + c * CHUNK, CHUNK)],
                             ebuf, esem)

    def drain(ebufs, mbuf, msem, esem):
        pltpu.make_async_copy(tm_hbm.at[pl.ds(0, CHUNK)], mbuf, msem).wait()
        for src, ebuf in zip(rows_hbm, ebufs):
            pltpu.make_async_copy(src.at[pl.ds(0, CHUNK)], ebuf,
                                  esem).wait()

    def compute(ebufs, mbuf, acc):
        eb_h, eb_t, eb_nh, eb_nt, eb_r, eb_nr = ebufs

        def sample(s, acc):
            zeros = [jnp.zeros((L,), jnp.float32) for _ in range(EV)]

            def ebody(eo, carry):
                ph, pt, nh, nt = carry
                hv = eb_h[s, pl.ds(eo * L, L)]
                tv = eb_t[s, pl.ds(eo * L, L)]
                nhv = eb_nh[s, pl.ds(eo * L, L)]
                ntv = eb_nt[s, pl.ds(eo * L, L)]
                off0 = eo * (L * REL_SIZE)
                for el in range(L):
                    m = [mbuf[s, pl.ds(off0 + el * REL_SIZE + j * L, L)]
                         for j in range(EV)]
                    hb = jnp.full((L,), hv[el])
                    tb = jnp.full((L,), tv[el])
                    nhb = jnp.full((L,), nhv[el])
                    ntb = jnp.full((L,), ntv[el])
                    ph = tuple(ph[j] + hb * m[j] for j in range(EV))
                    pt = tuple(pt[j] + tb * m[j] for j in range(EV))
                    nh = tuple(nh[j] + nhb * m[j] for j in range(EV))
                    nt = tuple(nt[j] + ntb * m[j] for j in range(EV))
                return ph, pt, nh, nt

            init = (tuple(zeros), tuple(zeros), tuple(zeros), tuple(zeros))
            ph, pt, nh, nt = lax.fori_loop(0, EV, ebody, init)

            phn = _l2n(list(ph), ii)
            ptn = _l2n(list(pt), ii)
            nhn = _l2n(list(nh), ii)
            ntn = _l2n(list(nt), ii)
            prn = _l2n([eb_r[s, pl.ds(j * L, L)] for j in range(EV)], ii)
            nrn = _l2n([eb_nr[s, pl.ds(j * L, L)] for j in range(EV)], ii)

            pd = jnp.abs(phn[0] + prn[0] - ptn[0])
            nd = jnp.abs(nhn[0] + nrn[0] - ntn[0])
            for j in range(1, EV):
                pd = pd + jnp.abs(phn[j] + prn[j] - ptn[j])
                nd = nd + jnp.abs(nhn[j] + nrn[j] - ntn[j])
            p_sc = _lanesum(pd, ii)
            n_sc = _lanesum(nd, ii)
            return acc + jnp.maximum(p_sc - n_sc + MARGIN, 0.0)

        return lax.fori_loop(0, CHUNK, sample, acc)

    issue(0, ebufs0, mbuf0, msem0, esem0)

    def body(i, acc):
        c0 = 2 * i
        issue(c0 + 1, ebufs1, mbuf1, msem1, esem1)
        drain(ebufs0, mbuf0, msem0, esem0)
        acc = compute(ebufs0, mbuf0, acc)

        @pl.when(i < NCHUNK // 2 - 1)
        def _():
            issue(c0 + 2, ebufs0, mbuf0, msem0, esem0)

        drain(ebufs1, mbuf1, msem1, esem1)
        return compute(ebufs1, mbuf1, acc)

    acc = lax.fori_loop(0, NCHUNK // 2, body, jnp.zeros((L,), jnp.float32))
    outv[pl.ds(0, L)] = acc
    pltpu.sync_copy(outv, out_hbm.at[wid])


def kernel(ent_embeddings, rel_embeddings, transfer_matrix,
           pos_h, pos_t, pos_r, neg_h, neg_t, neg_r):
    pos_h, pos_t, pos_r, neg_h, neg_t, neg_r = (
        x.astype(jnp.int32) for x in (pos_h, pos_t, pos_r,
                                      neg_h, neg_t, neg_r))
    he = jnp.take(ent_embeddings, pos_h, axis=0)
    te = jnp.take(ent_embeddings, pos_t, axis=0)
    nhe = jnp.take(ent_embeddings, neg_h, axis=0)
    nte = jnp.take(ent_embeddings, neg_t, axis=0)
    re = jnp.take(rel_embeddings, pos_r, axis=0)
    nre = jnp.take(rel_embeddings, neg_r, axis=0)

    mesh = plsc.VectorSubcoreMesh(core_axis_name="c", subcore_axis_name="s")
    erows = pltpu.VMEM((CHUNK, ENT_SIZE), jnp.float32)
    run = pl.kernel(
        _sc_body,
        out_type=jax.ShapeDtypeStruct((NW, OUTW), jnp.float32),
        mesh=mesh,
        scratch_types=(
            [pltpu.VMEM((BPW,), jnp.int32)]
            + [erows] * 12
            + [pltpu.VMEM((CHUNK, ENT_SIZE * REL_SIZE), jnp.float32)] * 2
            + [pltpu.VMEM((OUTW,), jnp.float32)]
            + [pltpu.SemaphoreType.DMA] * 4
        ),
    )
    partials = run(transfer_matrix, he, te, nhe, nte, re, nre, pos_r)
    return jnp.sum(partials[:, 0]) / BATCH


# speedup vs baseline: 2.4808x; 1.3351x over previous
import jax
import jax.numpy as jnp
from jax import lax
from jax.experimental import pallas as pl
from jax.experimental.pallas import tpu as pltpu
from jax.experimental.pallas import tpu_sc as plsc

ENT_SIZE = 64
REL_SIZE = 64
MARGIN = 1.0
BATCH = 4096
NC, NS = 2, 16
NW = NC * NS
BPW = BATCH // NW
CHUNK = 8
NCHUNK = BPW // CHUNK
L = 16
EV = ENT_SIZE // L
OUTW = 128


def _rsqrt_nr(x):
    xh = x * 0.5
    i = lax.bitcast_convert_type(x, jnp.int32)
    i = jnp.int32(0x5F3759DF) - lax.shift_right_logical(i, 1)
    y = lax.bitcast_convert_type(i, jnp.float32)
    for _ in range(3):
        y = y * (1.5 - xh * y * y)
    return y


def _lanesum(x, ii):
    for sh in (8, 4, 2, 1):
        x = x + jnp.take(x, ii ^ sh)
    return x


def _l2n(vs, ii):
    ss = vs[0] * vs[0]
    for j in range(1, EV):
        ss = ss + vs[j] * vs[j]
    total = _lanesum(ss, ii)
    inv = _rsqrt_nr(jnp.maximum(total, 1e-12))
    return [v * inv for v in vs]


def _sc_body(tm_hbm, he_hbm, te_hbm, nhe_hbm, nte_hbm, re_hbm, nre_hbm,
             pr_hbm,
             out_hbm,
             iv_pr,
             eb_h, eb_t, eb_nh, eb_nt, eb_r, eb_nr,
             mbuf, outv,
             msem0, msem1, esem0, esem1):
    wid = lax.axis_index("s") * NC + lax.axis_index("c")
    base = wid * BPW
    ii = lax.iota(jnp.int32, L)

    ebufs = (eb_h, eb_t, eb_nh, eb_nt, eb_r, eb_nr)
    rows_hbm = (he_hbm, te_hbm, nhe_hbm, nte_hbm, re_hbm, nre_hbm)

    pltpu.sync_copy(pr_hbm.at[pl.ds(base, BPW)], iv_pr)

    def issue(c, half, msem, esem):
        rb = half * CHUNK
        pltpu.async_copy(tm_hbm.at[iv_pr.at[pl.ds(c * CHUNK, CHUNK)]],
                         mbuf.at[pl.ds(rb, CHUNK)], msem)
        for src, ebuf in zip(rows_hbm, ebufs):
            pltpu.async_copy(src.at[pl.ds(base + c * CHUNK, CHUNK)],
                             ebuf.at[pl.ds(rb, CHUNK)], esem)

    def drain(half, msem, esem):
        rb = half * CHUNK
        pltpu.make_async_copy(tm_hbm.at[pl.ds(0, CHUNK)],
                              mbuf.at[pl.ds(rb, CHUNK)], msem).wait()
        for src, ebuf in zip(rows_hbm, ebufs):
            pltpu.make_async_copy(src.at[pl.ds(0, CHUNK)],
                                  ebuf.at[pl.ds(rb, CHUNK)], esem).wait()

    def compute(rb, acc):
        def sample(s, acc):
            r = rb + s
            ph = [jnp.zeros((L,), jnp.float32) for _ in range(EV)]
            pt = [jnp.zeros((L,), jnp.float32) for _ in range(EV)]
            nh = [jnp.zeros((L,), jnp.float32) for _ in range(EV)]
            nt = [jnp.zeros((L,), jnp.float32) for _ in range(EV)]
            for eo in range(EV):
                hv = eb_h[r, pl.ds(eo * L, L)]
                tv = eb_t[r, pl.ds(eo * L, L)]
                nhv = eb_nh[r, pl.ds(eo * L, L)]
                ntv = eb_nt[r, pl.ds(eo * L, L)]
                off0 = eo * (L * REL_SIZE)
                for el in range(L):
                    m = [mbuf[r, pl.ds(off0 + el * REL_SIZE + j * L, L)]
                         for j in range(EV)]
                    hb = jnp.full((L,), hv[el])
                    tb = jnp.full((L,), tv[el])
                    nhb = jnp.full((L,), nhv[el])
                    ntb = jnp.full((L,), ntv[el])
                    for j in range(EV):
                        ph[j] = ph[j] + hb * m[j]
                        pt[j] = pt[j] + tb * m[j]
                        nh[j] = nh[j] + nhb * m[j]
                        nt[j] = nt[j] + ntb * m[j]

            phn = _l2n(ph, ii)
            ptn = _l2n(pt, ii)
            nhn = _l2n(nh, ii)
            ntn = _l2n(nt, ii)
            prn = _l2n([eb_r[r, pl.ds(j * L, L)] for j in range(EV)], ii)
            nrn = _l2n([eb_nr[r, pl.ds(j * L, L)] for j in range(EV)], ii)

            pd = jnp.abs(phn[0] + prn[0] - ptn[0])
            nd = jnp.abs(nhn[0] + nrn[0] - ntn[0])
            for j in range(1, EV):
                pd = pd + jnp.abs(phn[j] + prn[j] - ptn[j])
                nd = nd + jnp.abs(nhn[j] + nrn[j] - ntn[j])
            p_sc = _lanesum(pd, ii)
            n_sc = _lanesum(nd, ii)
            return acc + jnp.maximum(p_sc - n_sc + MARGIN, 0.0)

        return lax.fori_loop(0, CHUNK, sample, acc)

    issue(0, 0, msem0, esem0)

    def body(c, acc):
        par = c & 1

        @pl.when(par == 0)
        def _():
            @pl.when(c < NCHUNK - 1)
            def _():
                issue(c + 1, 1, msem1, esem1)
            drain(0, msem0, esem0)

        @pl.when(par == 1)
        def _():
            @pl.when(c < NCHUNK - 1)
            def _():
                issue(c + 1, 0, msem0, esem0)
            drain(1, msem1, esem1)

        return compute(par * CHUNK, acc)

    acc = lax.fori_loop(0, NCHUNK, body, jnp.zeros((L,), jnp.float32))
    outv[pl.ds(0, L)] = acc
    pltpu.sync_copy(outv, out_hbm.at[wid])


def kernel(ent_embeddings, rel_embeddings, transfer_matrix,
           pos_h, pos_t, pos_r, neg_h, neg_t, neg_r):
    pos_h, pos_t, pos_r, neg_h, neg_t, neg_r = (
        x.astype(jnp.int32) for x in (pos_h, pos_t, pos_r,
                                      neg_h, neg_t, neg_r))
    he = jnp.take(ent_embeddings, pos_h, axis=0)
    te = jnp.take(ent_embeddings, pos_t, axis=0)
    nhe = jnp.take(ent_embeddings, neg_h, axis=0)
    nte = jnp.take(ent_embeddings, neg_t, axis=0)
    re = jnp.take(rel_embeddings, pos_r, axis=0)
    nre = jnp.take(rel_embeddings, neg_r, axis=0)

    mesh = plsc.VectorSubcoreMesh(core_axis_name="c", subcore_axis_name="s")
    erows = pltpu.VMEM((2 * CHUNK, ENT_SIZE), jnp.float32)
    run = pl.kernel(
        _sc_body,
        out_type=jax.ShapeDtypeStruct((NW, OUTW), jnp.float32),
        mesh=mesh,
        scratch_types=(
            [pltpu.VMEM((BPW,), jnp.int32)]
            + [erows] * 6
            + [pltpu.VMEM((2 * CHUNK, ENT_SIZE * REL_SIZE), jnp.float32)]
            + [pltpu.VMEM((OUTW,), jnp.float32)]
            + [pltpu.SemaphoreType.DMA] * 4
        ),
    )
    partials = run(transfer_matrix, he, te, nhe, nte, re, nre, pos_r)
    return jnp.sum(partials[:, 0]) / BATCH


# speedup vs baseline: 2.5482x; 1.0272x over previous
import jax
import jax.numpy as jnp
from jax import lax
from jax.experimental import pallas as pl
from jax.experimental.pallas import tpu as pltpu
from jax.experimental.pallas import tpu_sc as plsc

ENT_SIZE = 64
REL_SIZE = 64
MARGIN = 1.0
BATCH = 4096
NC, NS = 2, 16
NW = NC * NS
BPW = BATCH // NW
CHUNK = 8
NCHUNK = BPW // CHUNK
L = 16
EV = ENT_SIZE // L
OUTW = 128


def _rsqrt_nr(x):
    xh = x * 0.5
    i = lax.bitcast_convert_type(x, jnp.int32)
    i = jnp.int32(0x5F3759DF) - lax.shift_right_logical(i, 1)
    y = lax.bitcast_convert_type(i, jnp.float32)
    for _ in range(3):
        y = y * (1.5 - xh * y * y)
    return y


def _lanesum(x, ii):
    for sh in (8, 4, 2, 1):
        x = x + jnp.take(x, ii ^ sh)
    return x


def _l2n(vs, ii):
    ss = vs[0] * vs[0]
    for j in range(1, EV):
        ss = ss + vs[j] * vs[j]
    total = _lanesum(ss, ii)
    inv = _rsqrt_nr(jnp.maximum(total, 1e-12))
    return [v * inv for v in vs]


def _sc_body(tm_hbm, erow_hbm, rrow_hbm,
             pr_hbm,
             out_hbm,
             iv_pr,
             eb_h, eb_t, eb_nh, eb_nt, eb_r, eb_nr,
             mbuf, outv,
             msem0, msem1, esem0, esem1):
    wid = lax.axis_index("s") * NC + lax.axis_index("c")
    base = wid * BPW
    ii = lax.iota(jnp.int32, L)

    ebufs = (eb_h, eb_t, eb_nh, eb_nt, eb_r, eb_nr)
    srcs = ((erow_hbm, 0), (erow_hbm, BATCH), (erow_hbm, 2 * BATCH),
            (erow_hbm, 3 * BATCH), (rrow_hbm, 0), (rrow_hbm, BATCH))

    pltpu.sync_copy(pr_hbm.at[pl.ds(base, BPW)], iv_pr)

    def issue(c, half, msem, esem):
        rb = half * CHUNK
        pltpu.async_copy(tm_hbm.at[iv_pr.at[pl.ds(c * CHUNK, CHUNK)]],
                         mbuf.at[pl.ds(rb, CHUNK)], msem)
        for (src, off), ebuf in zip(srcs, ebufs):
            pltpu.async_copy(src.at[pl.ds(off + base + c * CHUNK, CHUNK)],
                             ebuf.at[pl.ds(rb, CHUNK)], esem)

    def drain(half, msem, esem):
        rb = half * CHUNK
        pltpu.make_async_copy(tm_hbm.at[pl.ds(0, CHUNK)],
                              mbuf.at[pl.ds(rb, CHUNK)], msem).wait()
        for (src, _), ebuf in zip(srcs, ebufs):
            pltpu.make_async_copy(src.at[pl.ds(0, CHUNK)],
                                  ebuf.at[pl.ds(rb, CHUNK)], esem).wait()

    def compute(rb, acc):
        def sample(s, acc):
            r = rb + s
            ph = [jnp.zeros((L,), jnp.float32) for _ in range(EV)]
            pt = [jnp.zeros((L,), jnp.float32) for _ in range(EV)]
            nh = [jnp.zeros((L,), jnp.float32) for _ in range(EV)]
            nt = [jnp.zeros((L,), jnp.float32) for _ in range(EV)]
            for eo in range(EV):
                hv = eb_h[r, pl.ds(eo * L, L)]
                tv = eb_t[r, pl.ds(eo * L, L)]
                nhv = eb_nh[r, pl.ds(eo * L, L)]
                ntv = eb_nt[r, pl.ds(eo * L, L)]
                off0 = eo * (L * REL_SIZE)
                for el in range(L):
                    m = [mbuf[r, pl.ds(off0 + el * REL_SIZE + j * L, L)]
                         for j in range(EV)]
                    hb = jnp.full((L,), hv[el])
                    tb = jnp.full((L,), tv[el])
                    nhb = jnp.full((L,), nhv[el])
                    ntb = jnp.full((L,), ntv[el])
                    for j in range(EV):
                        ph[j] = ph[j] + hb * m[j]
                        pt[j] = pt[j] + tb * m[j]
                        nh[j] = nh[j] + nhb * m[j]
                        nt[j] = nt[j] + ntb * m[j]

            phn = _l2n(ph, ii)
            ptn = _l2n(pt, ii)
            nhn = _l2n(nh, ii)
            ntn = _l2n(nt, ii)
            prn = _l2n([eb_r[r, pl.ds(j * L, L)] for j in range(EV)], ii)
            nrn = _l2n([eb_nr[r, pl.ds(j * L, L)] for j in range(EV)], ii)

            pd = jnp.abs(phn[0] + prn[0] - ptn[0])
            nd = jnp.abs(nhn[0] + nrn[0] - ntn[0])
            for j in range(1, EV):
                pd = pd + jnp.abs(phn[j] + prn[j] - ptn[j])
                nd = nd + jnp.abs(nhn[j] + nrn[j] - ntn[j])
            p_sc = _lanesum(pd, ii)
            n_sc = _lanesum(nd, ii)
            return acc + jnp.maximum(p_sc - n_sc + MARGIN, 0.0)

        return lax.fori_loop(0, CHUNK, sample, acc)

    issue(0, 0, msem0, esem0)

    def body(c, acc):
        par = c & 1

        @pl.when(par == 0)
        def _():
            @pl.when(c < NCHUNK - 1)
            def _():
                issue(c + 1, 1, msem1, esem1)
            drain(0, msem0, esem0)

        @pl.when(par == 1)
        def _():
            @pl.when(c < NCHUNK - 1)
            def _():
                issue(c + 1, 0, msem0, esem0)
            drain(1, msem1, esem1)

        return compute(par * CHUNK, acc)

    acc = lax.fori_loop(0, NCHUNK, body, jnp.zeros((L,), jnp.float32))
    outv[pl.ds(0, L)] = acc
    pltpu.sync_copy(outv, out_hbm.at[wid])


def kernel(ent_embeddings, rel_embeddings, transfer_matrix,
           pos_h, pos_t, pos_r, neg_h, neg_t, neg_r):
    pos_h, pos_t, pos_r, neg_h, neg_t, neg_r = (
        x.astype(jnp.int32) for x in (pos_h, pos_t, pos_r,
                                      neg_h, neg_t, neg_r))
    erows = jnp.take(ent_embeddings,
                     jnp.concatenate([pos_h, pos_t, neg_h, neg_t]), axis=0)
    rrows = jnp.take(rel_embeddings,
                     jnp.concatenate([pos_r, neg_r]), axis=0)

    mesh = plsc.VectorSubcoreMesh(core_axis_name="c", subcore_axis_name="s")
    ebuf_t = pltpu.VMEM((2 * CHUNK, ENT_SIZE), jnp.float32)
    run = pl.kernel(
        _sc_body,
        out_type=jax.ShapeDtypeStruct((NW, OUTW), jnp.float32),
        mesh=mesh,
        scratch_types=(
            [pltpu.VMEM((BPW,), jnp.int32)]
            + [ebuf_t] * 6
            + [pltpu.VMEM((2 * CHUNK, ENT_SIZE * REL_SIZE), jnp.float32)]
            + [pltpu.VMEM((OUTW,), jnp.float32)]
            + [pltpu.SemaphoreType.DMA] * 4
        ),
    )
    partials = run(transfer_matrix, erows, rrows, pos_r)
    return jnp.sum(partials[:, 0]) / BATCH
